# Initial kernel scaffold; baseline (speedup 1.0000x reference)
import jax
import jax.numpy as jnp
from jax.experimental import pallas as pl


def kernel(x, labels, centers):
    raise NotImplementedError("write your pallas kernel here")


# speedup vs baseline: 1.1071x; 1.1071x over previous
import functools

import jax
import jax.numpy as jnp
from jax import lax
from jax.experimental import pallas as pl
from jax.experimental.pallas import tpu as pltpu
from jax.experimental.pallas import tpu_sc as plsc

_BATCH = 16384
_FEAT = 128
_NC = 2
_NS = 16
_NW = _NC * _NS
_ROWS_PER_W = _BATCH // _NW
_CHUNK = 128
_NCHUNK = _ROWS_PER_W // _CHUNK
_LANES = 16
_VPF = _FEAT // _LANES


def _make_sc_loss():
    mesh = plsc.VectorSubcoreMesh(core_axis_name="c", subcore_axis_name="s")

    @functools.partial(
        pl.kernel,
        mesh=mesh,
        compiler_params=pltpu.CompilerParams(needs_layout_passes=False),
        out_type=jax.ShapeDtypeStruct((_NW, _LANES), jnp.float32),
        scratch_types=[
            pltpu.VMEM((_NCHUNK, _CHUNK), jnp.int32),
            pltpu.VMEM((2, _CHUNK, _FEAT), jnp.float32),
            pltpu.VMEM((2, _CHUNK, _FEAT), jnp.float32),
            pltpu.VMEM((_LANES,), jnp.float32),
            pltpu.SemaphoreType.DMA,
            pltpu.SemaphoreType.DMA,
            pltpu.SemaphoreType.DMA,
            pltpu.SemaphoreType.DMA,
        ],
    )
    def sc_loss(x_hbm, lab_hbm, cen_hbm, out_hbm,
                idx_v, xb, cb, res_v, sx0, sx1, sc0, sc1):
        wid = lax.axis_index("s") * _NC + lax.axis_index("c")
        pltpu.sync_copy(lab_hbm.at[wid], idx_v)
        sems_x = (sx0, sx1)
        sems_c = (sc0, sc1)
        hx = {}
        hc = {}

        def start(i):
            b = i % 2
            hx[i] = pltpu.async_copy(x_hbm.at[wid, i], xb.at[b], sems_x[b])
            hc[i] = pltpu.async_copy(cen_hbm.at[idx_v.at[i]], cb.at[b], sems_c[b])

        start(0)
        total = jnp.float32(0.0)
        for i in range(_NCHUNK):
            if i + 1 < _NCHUNK:
                start(i + 1)
            hx[i].wait()
            hc[i].wait()
            b = i % 2

            def row_body(r, tot, b=b):
                acc = jnp.zeros((_LANES,), jnp.float32)
                for f in range(_VPF):
                    xv = xb[b, r, pl.ds(_LANES * f, _LANES)]
                    cv = cb[b, r, pl.ds(_LANES * f, _LANES)]
                    d = xv - cv
                    acc = acc + d * d
                dist = plsc.cumsum(acc)[_LANES - 1]
                dist = jnp.minimum(jnp.maximum(dist, jnp.float32(1e-12)),
                                   jnp.float32(1e12))
                return tot + dist

            total = lax.fori_loop(0, _CHUNK, row_body, total)
        res_v[...] = jnp.full((_LANES,), total, jnp.float32)
        pltpu.sync_copy(res_v, out_hbm.at[wid])

    return sc_loss


_sc_loss = _make_sc_loss()


def kernel(x, labels, centers):
    x4 = x.reshape(_NW, _NCHUNK, _CHUNK, _FEAT)
    lab3 = labels.astype(jnp.int32).reshape(_NW, _NCHUNK, _CHUNK)
    partials = _sc_loss(x4, lab3, centers)
    return jnp.sum(partials[:, 0]) / jnp.float32(_BATCH)
